# Initial kernel scaffold; baseline (speedup 1.0000x reference)
import jax
import jax.numpy as jnp
from jax.experimental import pallas as pl


def kernel(pred_seq_logits, last_pred_logits, last_seq_mask, infer_mask, task_seq):
    raise NotImplementedError("write your pallas kernel here")


# speedup vs baseline: 1.3538x; 1.3538x over previous
import jax
import jax.numpy as jnp
from jax import lax
from jax.experimental import pallas as pl
from jax.experimental.pallas import tpu as pltpu

_B, _NS, _L, _P, _V = 8, 2, 128, 8, 32768
_Lp = _L - _P
_R = _B * _NS
_LB = 8
_NLB = _Lp // _LB
_K = _Lp // 2
_MASK_TOKEN = 3


def _row_from_col(col_mat, eye, ones_row):
    return lax.dot_general(
        ones_row, col_mat * eye, (((1,), (0,)), ((), ())),
        precision=lax.Precision.HIGHEST,
        preferred_element_type=jnp.float32)


def _body(pred_ref, last_ref, mask_ref, ens_ref, mb_ref, sm_ref, sc_col, ix_col):
    lb = pl.program_id(1)

    @pl.when(lb == 0)
    def _():
        sc_col[...] = jnp.full((_L, 1), -1.0, jnp.float32)
        ix_col[...] = jnp.zeros((_L, 1), jnp.float32)

    a = pred_ref[0]
    b = last_ref[0]
    m = mask_ref[0, pl.ds(lb * _LB, _LB), :]
    ens = jnp.where(m != 0.0, (a + b) * 0.5, b)
    ens_ref[0] = ens

    mx = jnp.max(ens, axis=1, keepdims=True)
    s = jnp.sum(jnp.exp(ens - mx), axis=1, keepdims=True)
    ii = lax.broadcasted_iota(jnp.int32, (_LB, _V), 1)
    am = jnp.min(jnp.where(ens == mx, ii, _V), axis=1, keepdims=True)
    sc_col[pl.ds(lb * _LB, _LB), :] = s
    ix_col[pl.ds(lb * _LB, _LB), :] = am.astype(jnp.float32)

    @pl.when(lb == _NLB - 1)
    def _():
        bi = lax.broadcasted_iota(jnp.int32, (_L, _L), 0)
        bj = lax.broadcasted_iota(jnp.int32, (_L, _L), 1)
        eye = (bi == bj).astype(jnp.float32)
        ones_row = jnp.ones((1, _L), jnp.float32)
        si = jnp.broadcast_to(sc_col[...], (_L, _L))
        s_row = _row_from_col(si, eye, ones_row)
        i_row = _row_from_col(jnp.broadcast_to(ix_col[...], (_L, _L)), eye,
                              ones_row)
        sj = jnp.broadcast_to(s_row, (_L, _L))
        cmp = ((si > sj) | ((si == sj) & (bi < bj))).astype(jnp.float32)
        rank = lax.dot_general(ones_row, cmp, (((1,), (0,)), ((), ())),
                               precision=lax.Precision.HIGHEST,
                               preferred_element_type=jnp.float32)
        maskv = rank < float(_K)
        seq = i_row.astype(jnp.int32)
        mb_ref[0] = jnp.where(maskv, _MASK_TOKEN, seq)
        sm_ref[0] = maskv.astype(jnp.int32)


def kernel(pred_seq_logits, last_pred_logits, last_seq_mask, infer_mask, task_seq):
    pred = pred_seq_logits.reshape(_R, _L, _V)
    last = last_pred_logits.reshape(_R, _Lp, _V)
    maskf = last_seq_mask.reshape(_R, _Lp, 1).astype(jnp.float32)
    ens, mb, sm = pl.pallas_call(
        _body,
        grid=(_R, _NLB),
        in_specs=[
            pl.BlockSpec((1, _LB, _V), lambda bn, lb: (bn, lb + 1, 0)),
            pl.BlockSpec((1, _LB, _V), lambda bn, lb: (bn, lb, 0)),
            pl.BlockSpec((1, _Lp, 1), lambda bn, lb: (bn, 0, 0)),
        ],
        out_specs=[
            pl.BlockSpec((1, _LB, _V), lambda bn, lb: (bn, lb, 0)),
            pl.BlockSpec((1, 1, _L), lambda bn, lb: (bn, 0, 0)),
            pl.BlockSpec((1, 1, _L), lambda bn, lb: (bn, 0, 0)),
        ],
        out_shape=[
            jax.ShapeDtypeStruct((_R, _Lp, _V), jnp.float32),
            jax.ShapeDtypeStruct((_R, 1, _L), jnp.int32),
            jax.ShapeDtypeStruct((_R, 1, _L), jnp.int32),
        ],
        scratch_shapes=[pltpu.VMEM((_L, 1), jnp.float32),
                        pltpu.VMEM((_L, 1), jnp.float32)],
    )(pred, last, maskf)
    seq_mask = (sm.reshape(_R, _L)[:, :_Lp] != 0).reshape(_B, _NS, _Lp)
    body = mb.reshape(_R, _L)[:, :_Lp].reshape(_B, _NS, _Lp)
    prompt = jnp.broadcast_to(task_seq.reshape(1, 1, _P),
                              (_B, _NS, _P)).astype(jnp.int32)
    masked_seq = jnp.concatenate([prompt, body], axis=2)
    return masked_seq, seq_mask, ens.reshape(_B, _NS, _Lp, _V)
